# baseline (speedup 1.0000x reference)
import jax
import jax.numpy as jnp
from jax.experimental import pallas as pl
from jax.experimental.pallas import tpu as pltpu
from jax.experimental.pallas import tpu_sc as plsc

_ROWS = 27278
_COLS = 1024
_BLOCK_ROWS = 2048
_SC_BLOCKS = 2
_SC_ROWS = _SC_BLOCKS * _BLOCK_ROWS
_TC_STEPS = (_ROWS - _SC_ROWS + _BLOCK_ROWS - 1) // _BLOCK_ROWS

_SC_ELEMS = _SC_ROWS * _COLS
_SC_CHUNK = 4096
_VEC = 16


def _fold(x):
    return jnp.sum(x.reshape(_BLOCK_ROWS // 8, 8, _COLS), axis=0)


def _tc_body(o_ref, t_ref, res_ref, acc_ref, cnt_ref):
    i = pl.program_id(0)

    @pl.when(i == 0)
    def _init():
        acc_ref[...] = jnp.zeros_like(acc_ref)
        cnt_ref[...] = jnp.zeros_like(cnt_ref)

    o = o_ref[...]
    t = t_ref[...]
    m = t != -1.0
    d = o - t

    @pl.when(i < _TC_STEPS - 1)
    def _full():
        acc_ref[...] += _fold(jnp.where(m, d * d, 0.0))
        cnt_ref[...] += _fold(m.astype(jnp.float32))

    @pl.when(i == _TC_STEPS - 1)
    def _tail():
        rows_left = _ROWS - _SC_ROWS - (_TC_STEPS - 1) * _BLOCK_ROWS
        valid = jax.lax.broadcasted_iota(
            jnp.int32, (_BLOCK_ROWS, _COLS), 0) < rows_left
        mv = jnp.logical_and(m, valid)
        acc_ref[...] += _fold(jnp.where(mv, d * d, 0.0))
        cnt_ref[...] += _fold(mv.astype(jnp.float32))
        res_ref[0, 0] = jnp.sum(acc_ref[...])
        res_ref[0, 1] = jnp.sum(cnt_ref[...])


def _tc_partial(o_t, t_t):
    spec = pl.BlockSpec((_BLOCK_ROWS, _COLS), lambda i: (i + _SC_BLOCKS, 0))
    res = pl.pallas_call(
        _tc_body,
        grid=(_TC_STEPS,),
        in_specs=[spec, spec],
        out_specs=pl.BlockSpec(memory_space=pltpu.SMEM),
        out_shape=jax.ShapeDtypeStruct((1, 2), jnp.float32),
        scratch_shapes=[
            pltpu.VMEM((8, _COLS), jnp.float32),
            pltpu.VMEM((8, _COLS), jnp.float32),
        ],
    )(o_t, t_t)
    return res[0, 0], res[0, 1]


def _sc_partial(o_t, t_t):
    o1 = o_t.reshape(-1)
    t1 = t_t.reshape(-1)
    mesh = plsc.VectorSubcoreMesh(core_axis_name="c", subcore_axis_name="s")
    n_units = mesh.num_cores * mesh.num_subcores

    @pl.kernel(
        out_type=(
            jax.ShapeDtypeStruct((n_units, _VEC), jnp.float32),
            jax.ShapeDtypeStruct((n_units, _VEC), jnp.float32),
        ),
        mesh=mesh,
        scratch_types=[
            pltpu.VMEM((_VEC,), jnp.float32),
            pltpu.VMEM((_VEC,), jnp.float32),
            pltpu.SemaphoreType.DMA,
        ],
    )
    def sc_kernel(o_hbm, t_hbm, sum_out, cnt_out, sacc, cacc, sem):
        sacc[...] = jnp.zeros((_VEC,), jnp.float32)
        cacc[...] = jnp.zeros((_VEC,), jnp.float32)

        def body(o_vmem, t_vmem):
            @pl.loop(0, _SC_CHUNK, step=_VEC)
            def _(c):
                o = o_vmem.at[pl.ds(c, _VEC)][...]
                t = t_vmem.at[pl.ds(c, _VEC)][...]
                m = t != -1.0
                d = o - t
                sacc[...] += jnp.where(m, d * d, 0.0)
                cacc[...] += jnp.where(m, 1.0, 0.0)

        pltpu.emit_pipeline(
            body,
            grid=(_SC_ELEMS // _SC_CHUNK,),
            in_specs=[
                pl.BlockSpec((_SC_CHUNK,), index_map=lambda i: (i,)),
                pl.BlockSpec((_SC_CHUNK,), index_map=lambda i: (i,)),
            ],
            core_axis_name=("c", "s"),
            dimension_semantics=(pltpu.PARALLEL,),
        )(o_hbm, t_hbm)

        c = jax.lax.axis_index("c")
        s = jax.lax.axis_index("s")
        idx = c * mesh.num_subcores + s
        pltpu.sync_copy(sacc, sum_out.at[idx])
        pltpu.sync_copy(cacc, cnt_out.at[idx])

    sums, cnts = sc_kernel(o1, t1)
    return jnp.sum(sums), jnp.sum(cnts)


def kernel(output, target):
    o_t = output.T
    t_t = target.T
    tc_sum, tc_cnt = _tc_partial(o_t, t_t)
    sc_sum, sc_cnt = _sc_partial(o_t, t_t)
    return (tc_sum + sc_sum) / (tc_cnt + sc_cnt)


# speedup vs baseline: 4.2656x; 4.2656x over previous
import jax
import jax.numpy as jnp
from jax.experimental import pallas as pl
from jax.experimental.pallas import tpu as pltpu

_ROWS = 27278
_COLS = 1024
_BLOCK_ROWS = 2048
_STEPS = (_ROWS + _BLOCK_ROWS - 1) // _BLOCK_ROWS


def _fold(x):
    return jnp.sum(x.reshape(_BLOCK_ROWS // 8, 8, _COLS), axis=0)


def _masked_mse_body(o_ref, t_ref, res_ref, acc_ref, cnt_ref):
    i = pl.program_id(0)

    @pl.when(i == 0)
    def _init():
        acc_ref[...] = jnp.zeros_like(acc_ref)
        cnt_ref[...] = jnp.zeros_like(cnt_ref)

    o = o_ref[...]
    t = t_ref[...]
    m = t != -1.0
    d = o - t

    @pl.when(i < _STEPS - 1)
    def _full():
        acc_ref[...] += _fold(jnp.where(m, d * d, 0.0))
        cnt_ref[...] += _fold(m.astype(jnp.float32))

    @pl.when(i == _STEPS - 1)
    def _tail():
        rows_left = _ROWS - (_STEPS - 1) * _BLOCK_ROWS
        valid = jax.lax.broadcasted_iota(
            jnp.int32, (_BLOCK_ROWS, _COLS), 0) < rows_left
        mv = jnp.logical_and(m, valid)
        acc_ref[...] += _fold(jnp.where(mv, d * d, 0.0))
        cnt_ref[...] += _fold(mv.astype(jnp.float32))
        res_ref[0, 0] = jnp.sum(acc_ref[...]) / jnp.sum(cnt_ref[...])


def kernel(output, target):
    spec = pl.BlockSpec((_BLOCK_ROWS, _COLS), lambda i: (i, 0))
    res = pl.pallas_call(
        _masked_mse_body,
        grid=(_STEPS,),
        in_specs=[spec, spec],
        out_specs=pl.BlockSpec(memory_space=pltpu.SMEM),
        out_shape=jax.ShapeDtypeStruct((1, 1), jnp.float32),
        scratch_shapes=[
            pltpu.VMEM((8, _COLS), jnp.float32),
            pltpu.VMEM((8, _COLS), jnp.float32),
        ],
    )(output.T, target.T)
    return res.reshape(())
